# Initial kernel scaffold; baseline (speedup 1.0000x reference)
import jax
import jax.numpy as jnp
from jax.experimental import pallas as pl


def kernel(x, edge_index, batch, params):
    raise NotImplementedError("write your pallas kernel here")


# speedup vs baseline: 11.5383x; 11.5383x over previous
import functools
import jax
import jax.numpy as jnp
from jax import lax
from jax.experimental import pallas as pl
from jax.experimental.pallas import tpu as pltpu
from jax.experimental.pallas import tpu_sc as plsc

N = 50000
NPAD = 50048
DUMP = NPAD - 1
E = 800000
EPAD = 819200
HEADS = 4
HID = 16
PROJ = 64
NUM_CLASSES = 14
NUM_GRAPHS = 256
EPS = 1e-5

NTILES = 32
EPW = EPAD // NTILES
CHUNK = 1024
NCHUNK = EPW // CHUNK
RPT = NPAD // 16
BLK = 3128
PBLK = 2000

_mesh = plsc.VectorSubcoreMesh(core_axis_name="c", subcore_axis_name="s")


def _g16(v, idx):
    dn = lax.GatherDimensionNumbers(
        offset_dims=(), collapsed_slice_dims=(0,), start_index_map=(0,))
    return lax.gather(v, idx[:, None], dn, (1,),
                      mode=lax.GatherScatterMode.PROMISE_IN_BOUNDS)


def _lane(v, i):
    idx = lax.iota(jnp.int32, 16) * 0 + i
    return _g16(v, idx)


@functools.partial(
    pl.kernel, mesh=_mesh,
    compiler_params=pltpu.CompilerParams(use_tc_tiling_on_sc=False),
    out_type=(jax.ShapeDtypeStruct((EPAD, 16), jnp.float32),
              jax.ShapeDtypeStruct((NPAD, 16), jnp.float32),
              jax.ShapeDtypeStruct((NPAD, 16), jnp.float32)),
    scratch_types=[
        pltpu.VMEM((CHUNK,), jnp.int32),
        pltpu.VMEM((CHUNK,), jnp.int32),
        pltpu.VMEM((CHUNK, 16), jnp.float32),
        pltpu.VMEM((CHUNK, 16), jnp.float32),
        pltpu.VMEM((CHUNK, 16), jnp.float32),
        pltpu.VMEM_SHARED((NPAD, 16), jnp.float32),
        pltpu.SemaphoreType.DMA,
        pltpu.SemaphoreType.DMA,
    ])
def _sc_edge_a(src_h, dst_h, a_h,
               e_h, s0_h, s1_h,
               srcv, dstv, gs, gd, ev, ssum, sem1, sem2):
    cid = lax.axis_index("c")
    sid = lax.axis_index("s")
    wid = sid * 2 + cid

    def zi(i, _):
        ev[i, :] = jnp.zeros((16,), jnp.float32)
        return 0

    lax.fori_loop(0, CHUNK, zi, 0)
    for off, sz in ((0, 1024), (1024, 1024), (2048, 1024), (3072, 56)):
        pltpu.sync_copy(ev.at[pl.ds(0, sz)],
                        ssum.at[pl.ds(sid * RPT + off, sz)])
    plsc.subcore_barrier()

    lanes = lax.iota(jnp.int32, 16)
    sh4 = jnp.bitwise_and(lanes + 4, 15)
    head_ok = lanes < 4

    def chunk(j, _):
        base = wid * EPW + j * CHUNK
        pltpu.sync_copy(src_h.at[pl.ds(base, CHUNK)], srcv)
        pltpu.sync_copy(dst_h.at[pl.ds(base, CHUNK)], dstv)
        pltpu.async_copy(a_h.at[srcv], gs, sem1).wait()
        pltpu.async_copy(a_h.at[dstv], gd, sem2).wait()

        def edge(i, _):
            s = gs[i, :]
            d = gd[i, :]
            al = s + _g16(d, sh4)
            al = jnp.where(al > 0, al, 0.2 * al)
            e = jnp.where(head_ok, jnp.exp(al), 0.0)
            ev[i, :] = e
            return 0

        lax.fori_loop(0, CHUNK, edge, 0)
        pltpu.sync_copy(ev, e_h.at[pl.ds(base, CHUNK)])
        for k in range(8):
            pltpu.sync_copy(ev.at[pl.ds(k * 128, 128)],
                            ssum.at[dstv.at[pl.ds(k * 128, 128)]], add=True)
        return 0

    lax.fori_loop(0, NCHUNK, chunk, 0)
    plsc.subcore_barrier()

    @pl.when(cid == 0)
    def _():
        pltpu.sync_copy(ssum.at[pl.ds(sid * RPT, RPT)],
                        s0_h.at[pl.ds(sid * RPT, RPT)])

    @pl.when(cid == 1)
    def _():
        pltpu.sync_copy(ssum.at[pl.ds(sid * RPT, RPT)],
                        s1_h.at[pl.ds(sid * RPT, RPT)])


NH = NPAD // 2
RPTB = NH // 16


def _make_sc_edge_b(hh, ps):
    lo = hh
    lon = ps * NH

    @functools.partial(
        pl.kernel, mesh=_mesh,
        compiler_params=pltpu.CompilerParams(use_tc_tiling_on_sc=False),
        out_type=(jax.ShapeDtypeStruct((NH, 16), jnp.float32),
                  jax.ShapeDtypeStruct((NH, 16), jnp.float32)),
        scratch_types=[
            pltpu.VMEM((CHUNK,), jnp.int32),
            pltpu.VMEM((CHUNK,), jnp.int32),
            pltpu.VMEM((CHUNK, 16), jnp.float32),
            pltpu.VMEM((CHUNK, 16), jnp.float32),
            pltpu.VMEM((CHUNK, 16), jnp.float32),
            pltpu.VMEM((CHUNK, 16), jnp.float32),
            pltpu.VMEM((CHUNK, 16), jnp.float32),
            pltpu.VMEM_SHARED((NH, 16), jnp.float32),
            pltpu.SemaphoreType.DMA,
            pltpu.SemaphoreType.DMA,
            pltpu.SemaphoreType.DMA,
        ])
    def _sc_edge_b(src_h, dst_h, e_h, s0_h, s1_h, ht_h,
                   m0_h, m1_h,
                   srcv, dstv, ev, g0, g1, hv, mv, acc,
                   sem1, sem2, sem3):
         cid = lax.axis_index("c")
         sid = lax.axis_index("s")
         wid = sid * 2 + cid

         def zi(i, _):
             ev[i, :] = jnp.zeros((16,), jnp.float32)
             return 0

         lax.fori_loop(0, CHUNK, zi, 0)
         for off, sz in ((0, 1024), (1024, 540)):
             pltpu.sync_copy(ev.at[pl.ds(0, sz)],
                             acc.at[pl.ds(sid * RPTB + off, sz)])
         plsc.subcore_barrier()
         lanes0 = lax.iota(jnp.int32, 16) * 0

         def chunk(j, _):
             base = wid * EPW + j * CHUNK
             pltpu.sync_copy(src_h.at[pl.ds(base, CHUNK)], srcv)
             pltpu.sync_copy(dst_h.at[pl.ds(base, CHUNK)], dstv)
             pltpu.sync_copy(e_h.at[pl.ds(base, CHUNK)], ev)
             pltpu.async_copy(s0_h.at[dstv], g0, sem1).wait()
             pltpu.async_copy(s1_h.at[dstv], g1, sem2).wait()
             pltpu.async_copy(ht_h.at[srcv], hv, sem3).wait()

             def edge(i, _):
                 dv = dstv[pl.ds((i // 16) * 16, 16)]
                 dd = _g16(dv, lanes0 + (i % 16))
                 okv = jnp.where((dd >= lon) & (dd < lon + NH), 1.0, 0.0)
                 w = ev[i, :] / (g0[i, :] + g1[i, :] + 1e-16) * okv
                 mv[i, :] = hv[i, :] * _lane(w, lo)
                 return 0

             lax.fori_loop(0, CHUNK, edge, 0)

             def cl(k2, _):
                 t = dstv[pl.ds(k2 * 16, 16)]
                 dstv[pl.ds(k2 * 16, 16)] = jnp.clip(t - lon, 0, NH - 1)
                 return 0

             lax.fori_loop(0, CHUNK // 16, cl, 0)
             for k in range(8):
                 pltpu.sync_copy(mv.at[pl.ds(k * 128, 128)],
                                 acc.at[dstv.at[pl.ds(k * 128, 128)]], add=True)
             return 0

         lax.fori_loop(0, NCHUNK, chunk, 0)
         plsc.subcore_barrier()

         @pl.when(cid == 0)
         def _():
             pltpu.sync_copy(acc.at[pl.ds(sid * RPTB, RPTB)],
                             m0_h.at[pl.ds(sid * RPTB, RPTB)])

         @pl.when(cid == 1)
         def _():
             pltpu.sync_copy(acc.at[pl.ds(sid * RPTB, RPTB)],
                             m1_h.at[pl.ds(sid * RPTB, RPTB)])

    return _sc_edge_b


_sc_edge_b = [[_make_sc_edge_b(i, q) for q in range(2)] for i in range(HEADS)]


def _k_in(x_pad, w, b):
    def body(x_r, w_r, b_r, o_r):
        o_r[...] = jnp.dot(x_r[...], w_r[...],
                           preferred_element_type=jnp.float32) + b_r[...]
    return pl.pallas_call(
        body,
        grid=(NPAD // BLK,),
        in_specs=[pl.BlockSpec((BLK, 128), lambda i: (i, 0)),
                  pl.BlockSpec((128, PROJ), lambda i: (0, 0)),
                  pl.BlockSpec((1, PROJ), lambda i: (0, 0))],
        out_specs=pl.BlockSpec((BLK, PROJ), lambda i: (i, 0)),
        out_shape=jax.ShapeDtypeStruct((NPAD, PROJ), jnp.float32),
    )(x_pad, w, b)


def _k_stage1(x, w, aa):
    def body(x_r, w_r, a_r, h0_r, h1_r, h2_r, h3_r, at_r):
        h = jnp.dot(x_r[...], w_r[...], preferred_element_type=jnp.float32)
        at_r[...] = jnp.dot(h, a_r[...], preferred_element_type=jnp.float32)
        h0_r[...] = h[:, 0:16]
        h1_r[...] = h[:, 16:32]
        h2_r[...] = h[:, 32:48]
        h3_r[...] = h[:, 48:64]
    return pl.pallas_call(
        body,
        grid=(NPAD // BLK,),
        in_specs=[pl.BlockSpec((BLK, PROJ), lambda i: (i, 0)),
                  pl.BlockSpec((PROJ, PROJ), lambda i: (0, 0)),
                  pl.BlockSpec((PROJ, 16), lambda i: (0, 0))],
        out_specs=[pl.BlockSpec((BLK, 16), lambda i: (i, 0))] * 5,
        out_shape=[jax.ShapeDtypeStruct((NPAD, 16), jnp.float32)] * 5,
    )(x, w, aa)


def _k_stage2(x, parts, b, g, bn):
    def body(x_r, p0_r, p1_r, p2_r, p3_r, p4_r, p5_r, p6_r, p7_r,
             bb_r, g_r, bn_r, o_r):
        msg = jnp.concatenate(
            [p0_r[...] + p1_r[...], p2_r[...] + p3_r[...],
             p4_r[...] + p5_r[...], p6_r[...] + p7_r[...]],
            axis=1) + bb_r[...]
        h = jnp.where(msg > 0, msg, jnp.exp(msg) - 1.0)
        r = x_r[...] + h
        mu = jnp.mean(r, axis=-1, keepdims=True)
        var = jnp.mean((r - mu) ** 2, axis=-1, keepdims=True)
        o_r[...] = (r - mu) / jnp.sqrt(var + EPS) * g_r[...] + bn_r[...]
    return pl.pallas_call(
        body,
        grid=(NPAD // BLK,),
        in_specs=[pl.BlockSpec((BLK, PROJ), lambda i: (i, 0))]
        + [pl.BlockSpec((BLK, 16), lambda i: (i, 0))] * 8
        + [pl.BlockSpec((1, PROJ), lambda i: (0, 0))] * 3,
        out_specs=pl.BlockSpec((BLK, PROJ), lambda i: (i, 0)),
        out_shape=jax.ShapeDtypeStruct((NPAD, PROJ), jnp.float32),
    )(x, *parts, b, g, bn)


def _k_pool1(x, w1, b1, w2, b2, batch3):
    def body(x_r, w1_r, b1_r, w2_r, b2_r, bt_r, sc_r, mx_r):
        i = pl.program_id(0)
        t = jnp.tanh(jnp.dot(x_r[...], w1_r[...],
                             preferred_element_type=jnp.float32) + b1_r[...])
        sc = jnp.dot(t, w2_r[...], preferred_element_type=jnp.float32)[:, 0] \
            + b2_r[0, 0]
        sc_r[0, 0, :] = sc
        m = bt_r[0, 0, :][:, None] == lax.broadcasted_iota(
            jnp.int32, (PBLK, NUM_GRAPHS), 1)
        bmax = jnp.max(jnp.where(m, sc[:, None], -jnp.inf), axis=0)

        @pl.when(i == 0)
        def _():
            mx_r[...] = jnp.full((1, NUM_GRAPHS), -jnp.inf, jnp.float32)

        mx_r[0, :] = jnp.maximum(mx_r[0, :], bmax)
    return pl.pallas_call(
        body,
        grid=(N // PBLK,),
        in_specs=[pl.BlockSpec((PBLK, PROJ), lambda i: (i, 0)),
                  pl.BlockSpec((PROJ, 32), lambda i: (0, 0)),
                  pl.BlockSpec((1, 32), lambda i: (0, 0)),
                  pl.BlockSpec((32, 1), lambda i: (0, 0)),
                  pl.BlockSpec((1, 1), lambda i: (0, 0)),
                  pl.BlockSpec((1, 1, PBLK), lambda i: (i, 0, 0))],
        out_specs=[pl.BlockSpec((1, 1, PBLK), lambda i: (i, 0, 0)),
                   pl.BlockSpec((1, NUM_GRAPHS), lambda i: (0, 0))],
        out_shape=[jax.ShapeDtypeStruct((N // PBLK, 1, PBLK), jnp.float32),
                   jax.ShapeDtypeStruct((1, NUM_GRAPHS), jnp.float32)],
    )(x, w1, b1, w2, b2, batch3)


def _k_pool2(sc3, batch3, smax):
    def body(sc_r, bt_r, mx_r, es_r, ss_r):
        i = pl.program_id(0)
        m = bt_r[0, 0, :][:, None] == lax.broadcasted_iota(
            jnp.int32, (PBLK, NUM_GRAPHS), 1)
        mxc = jnp.where(jnp.isfinite(mx_r[0, :]), mx_r[0, :], 0.0)
        mrow = jnp.dot(m.astype(jnp.float32), mxc[:, None],
                       preferred_element_type=jnp.float32)[:, 0]
        es = jnp.exp(sc_r[0, 0, :] - mrow)
        es_r[0, 0, :] = es

        @pl.when(i == 0)
        def _():
            ss_r[...] = jnp.zeros((1, NUM_GRAPHS), jnp.float32)

        ss_r[0, :] = ss_r[0, :] + jnp.dot(
            es[None, :], m.astype(jnp.float32),
            preferred_element_type=jnp.float32)[0, :]
    return pl.pallas_call(
        body,
        grid=(N // PBLK,),
        in_specs=[pl.BlockSpec((1, 1, PBLK), lambda i: (i, 0, 0)),
                  pl.BlockSpec((1, 1, PBLK), lambda i: (i, 0, 0)),
                  pl.BlockSpec((1, NUM_GRAPHS), lambda i: (0, 0))],
        out_specs=[pl.BlockSpec((1, 1, PBLK), lambda i: (i, 0, 0)),
                   pl.BlockSpec((1, NUM_GRAPHS), lambda i: (0, 0))],
        out_shape=[jax.ShapeDtypeStruct((N // PBLK, 1, PBLK), jnp.float32),
                   jax.ShapeDtypeStruct((1, NUM_GRAPHS), jnp.float32)],
    )(sc3, batch3, smax)


def _k_pool3(x, es3, batch3, ssum):
    def body(x_r, es_r, bt_r, ss_r, o_r):
        i = pl.program_id(0)
        m = (bt_r[0, 0, :][:, None] == lax.broadcasted_iota(
            jnp.int32, (PBLK, NUM_GRAPHS), 1)).astype(jnp.float32)
        srow = jnp.dot(m, ss_r[0, :][:, None],
                       preferred_element_type=jnp.float32)[:, 0]
        wts = es_r[0, 0, :] / (srow + 1e-16)

        @pl.when(i == 0)
        def _():
            o_r[...] = jnp.zeros((NUM_GRAPHS, PROJ), jnp.float32)

        o_r[...] = o_r[...] + jnp.dot((m * wts[:, None]).T, x_r[...],
                                      preferred_element_type=jnp.float32)
    return pl.pallas_call(
        body,
        grid=(N // PBLK,),
        in_specs=[pl.BlockSpec((PBLK, PROJ), lambda i: (i, 0)),
                  pl.BlockSpec((1, 1, PBLK), lambda i: (i, 0, 0)),
                  pl.BlockSpec((1, 1, PBLK), lambda i: (i, 0, 0)),
                  pl.BlockSpec((1, NUM_GRAPHS), lambda i: (0, 0))],
        out_specs=pl.BlockSpec((NUM_GRAPHS, PROJ), lambda i: (0, 0)),
        out_shape=jax.ShapeDtypeStruct((NUM_GRAPHS, PROJ), jnp.float32),
    )(x, es3, batch3, ssum)


def _k_cls(pooled, w1, b1, w2, b2):
    def body(p_r, w1_r, b1_r, w2_r, b2_r, o_r):
        h = jnp.dot(p_r[...], w1_r[...],
                    preferred_element_type=jnp.float32) + b1_r[...]
        h = 0.5 * h * (1.0 + lax.erf(h / jnp.sqrt(2.0).astype(jnp.float32)))
        o_r[...] = jnp.dot(h, w2_r[...],
                           preferred_element_type=jnp.float32) + b2_r[...]
    return pl.pallas_call(
        body,
        in_specs=[pl.BlockSpec((NUM_GRAPHS, PROJ), lambda: (0, 0)),
                  pl.BlockSpec((PROJ, 32), lambda: (0, 0)),
                  pl.BlockSpec((1, 32), lambda: (0, 0)),
                  pl.BlockSpec((32, NUM_CLASSES), lambda: (0, 0)),
                  pl.BlockSpec((1, NUM_CLASSES), lambda: (0, 0))],
        out_specs=pl.BlockSpec((NUM_GRAPHS, NUM_CLASSES), lambda: (0, 0)),
        out_shape=jax.ShapeDtypeStruct((NUM_GRAPHS, NUM_CLASSES), jnp.float32),
    )(pooled, w1, b1, w2, b2)


def kernel(x, edge_index, batch, params):
    p = params
    src = jnp.concatenate(
        [edge_index[0], jnp.full((EPAD - E,), DUMP, jnp.int32)])
    dst = jnp.concatenate(
        [edge_index[1], jnp.full((EPAD - E,), DUMP, jnp.int32)])
    x_pad = jnp.zeros((NPAD, 128), jnp.float32).at[:N].set(x)
    batch3 = batch.reshape(N // PBLK, 1, PBLK)

    xc = _k_in(x_pad, p['W_in'], p['b_in'].reshape(1, PROJ))
    rows = jnp.arange(PROJ)
    hcol = jnp.repeat(jnp.arange(HEADS), HID)
    for li in range(3):
        a_s = p['gat%d_as' % li].reshape(PROJ)
        a_d = p['gat%d_ad' % li].reshape(PROJ)
        aa = jnp.zeros((PROJ, 16), jnp.float32)
        aa = aa.at[rows, hcol].set(a_s).at[rows, 4 + hcol].set(a_d)
        h0, h1, h2, h3, at = _k_stage1(xc, p['gat%d_W' % li], aa)
        e_t, s0, s1 = _sc_edge_a(src, dst, at)
        parts = []
        for hh, ht in enumerate((h0, h1, h2, h3)):
            q0a, q0b = _sc_edge_b[hh][0](src, dst, e_t, s0, s1, ht)
            q1a, q1b = _sc_edge_b[hh][1](src, dst, e_t, s0, s1, ht)
            parts.append(jnp.concatenate([q0a, q1a], axis=0))
            parts.append(jnp.concatenate([q0b, q1b], axis=0))
        xc = _k_stage2(xc, parts,
                       p['gat%d_b' % li].reshape(1, PROJ),
                       p['norm%d_g' % li].reshape(1, PROJ),
                       p['norm%d_b' % li].reshape(1, PROJ))

    sc3, smax = _k_pool1(xc, p['Wp1'], p['bp1'].reshape(1, 32),
                         p['Wp2'], p['bp2'].reshape(1, 1), batch3)
    es3, ssum = _k_pool2(sc3, batch3, smax)
    pooled = _k_pool3(xc, es3, batch3, ssum)
    return _k_cls(pooled, p['Wc1'], p['bc1'].reshape(1, 32),
                  p['Wc2'], p['bc2'].reshape(1, NUM_CLASSES))
